# baseline (speedup 1.0000x reference)
import functools

import jax
import jax.numpy as jnp
from jax import lax
from jax.experimental import pallas as pl
from jax.experimental.pallas import tpu as pltpu
from jax.experimental.pallas import tpu_sc as plsc

TOP_K = 2
BLK = 256
HB = 1024


def _routing_body(x_ref, wg_ref, pos_ref, w_ref, emap_ref, amap_ref, *, n_e, blk):
    t_len = x_ref.shape[0]
    s = jax.lax.dot_general(wg_ref[...], x_ref[...], (((1,), (1,)), ((), ())),
                            preferred_element_type=jnp.float32)
    ei = jax.lax.broadcasted_iota(jnp.int32, s.shape, 0)
    v1 = jnp.max(s, axis=0, keepdims=True)
    i1 = jnp.min(jnp.where(s == v1, ei, n_e), axis=0, keepdims=True)
    s_m = jnp.where(ei == i1, -jnp.inf, s)
    v2 = jnp.max(s_m, axis=0, keepdims=True)
    i2 = jnp.min(jnp.where(s_m == v2, ei, n_e), axis=0, keepdims=True)
    p1 = 1.0 / (1.0 + jnp.exp(v2 - v1))
    w_ref[...] = jnp.concatenate([p1, 1.0 - p1], axis=0)

    e2d = jnp.concatenate([i1, i2], axis=0)
    tri = (jax.lax.broadcasted_iota(jnp.int32, (t_len, t_len), 0)
           <= jax.lax.broadcasted_iota(jnp.int32, (t_len, t_len), 1)
           ).astype(jnp.float32)
    ms = [(e2d == e).astype(jnp.float32) for e in range(n_e)]
    m_all = jnp.concatenate(ms, axis=0)
    c_all = jax.lax.dot_general(m_all, tri, (((1,), (0,)), ((), ())),
                                preferred_element_type=jnp.float32)

    rank = jnp.zeros_like(w_ref[...])
    counts = []
    for e in range(n_e):
        m = ms[e]
        c = c_all[2 * e:2 * e + 2]
        row0_tot = c[0:1, t_len - 1:t_len]
        carry = jnp.concatenate(
            [jnp.zeros((1, t_len), jnp.float32),
             jnp.broadcast_to(row0_tot, (1, t_len))], axis=0)
        rank = rank + m * (c - m + carry)
        counts.append(row0_tot + c[1:2, t_len - 1:t_len])

    blkstart = []
    run = jnp.zeros((1, 1), jnp.float32)
    for e in range(n_e):
        blkstart.append(run)
        run = run + jnp.floor((counts[e] + (blk - 1)) / blk)
    total_blocks = run

    astart = jnp.zeros_like(rank)
    for e in range(n_e):
        astart = astart + ms[e] * (blkstart[e] * blk)
    pos_ref[...] = (rank + astart).astype(jnp.int32)

    bi = jax.lax.broadcasted_iota(jnp.int32, (1, 128), 1)
    em = jnp.zeros((1, 128), jnp.int32)
    for e in range(n_e):
        bs = jnp.broadcast_to(blkstart[e].astype(jnp.int32), (1, 128))
        em = em + (bi >= bs).astype(jnp.int32)
    emap_ref[...] = em - 1
    amap_ref[...] = (
        bi < jnp.broadcast_to(total_blocks.astype(jnp.int32), (1, 128))
    ).astype(jnp.int32)


def _gemm_body(emap_ref, amap_ref, xs_ref, w1_ref, w2_ref, w3_ref, out_ref,
               acc_ref, *, blk):
    h = pl.program_id(0)
    b = pl.program_id(1)
    nh = pl.num_programs(0)

    @pl.when(amap_ref[0, b] > 0)
    def _compute():
        x = xs_ref[...].astype(jnp.float32)
        h1 = jax.lax.dot_general(x, w1_ref[0], (((1,), (1,)), ((), ())),
                                 preferred_element_type=jnp.float32)
        h2 = jax.lax.dot_general(x, w2_ref[0], (((1,), (1,)), ((), ())),
                                 preferred_element_type=jnp.float32)
        hidden = h1 * jax.lax.logistic(h1) * h2
        eo = jax.lax.dot_general(hidden, w3_ref[0], (((1,), (1,)), ((), ())),
                                 preferred_element_type=jnp.float32)

        sl = pl.ds(b * blk, blk)

        @pl.when(h == 0)
        def _init():
            acc_ref[sl, :] = eo

        @pl.when(h != 0)
        def _acc():
            acc_ref[sl, :] += eo

        @pl.when(h == nh - 1)
        def _fin():
            out_ref[...] = acc_ref[sl, :]


def kernel(x, Wg, W12, W3):
    b, t_len, d = x.shape
    n_e, h2, _ = W12.shape
    hdim = h2 // 2
    xt = x.reshape(b * t_len, d)
    tt = b * t_len
    na = TOP_K * tt
    blk = min(BLK, tt)
    nblk = na // blk + n_e
    npad = nblk * blk
    hb = min(HB, hdim)
    nh = hdim // hb

    pos2d, w2d, emap, amap = pl.pallas_call(
        functools.partial(_routing_body, n_e=n_e, blk=blk),
        out_shape=[
            jax.ShapeDtypeStruct((TOP_K, tt), jnp.int32),
            jax.ShapeDtypeStruct((TOP_K, tt), jnp.float32),
            jax.ShapeDtypeStruct((1, 128), jnp.int32),
            jax.ShapeDtypeStruct((1, 128), jnp.int32),
        ],
    )(xt, Wg)

    pos_flat = pos2d.reshape(na)
    xt_i32 = jax.lax.bitcast_convert_type(
        xt.astype(jnp.bfloat16).reshape(tt, d // 2, 2), jnp.int32)
    xs_i32 = _sc_dispatch(xt_i32, pos_flat, npad)
    xs = jax.lax.bitcast_convert_type(xs_i32, jnp.bfloat16).reshape(npad, d)
    out_s = pl.pallas_call(
        functools.partial(_gemm_body, blk=blk),
        grid_spec=pltpu.PrefetchScalarGridSpec(
            num_scalar_prefetch=2,
            grid=(nh, nblk),
            in_specs=[
                pl.BlockSpec((blk, d), lambda hh, bb, em, am: (bb, 0)),
                pl.BlockSpec((1, hb, d), lambda hh, bb, em, am: (em[0, bb], hh, 0)),
                pl.BlockSpec((1, hb, d),
                             lambda hh, bb, em, am, _nh=nh: (em[0, bb], _nh + hh, 0)),
                pl.BlockSpec((1, d, hb), lambda hh, bb, em, am: (em[0, bb], 0, hh)),
            ],
            out_specs=pl.BlockSpec((blk, d), lambda hh, bb, em, am: (bb, 0)),
            scratch_shapes=[pltpu.VMEM((npad, d), jnp.float32)],
        ),
        out_shape=jax.ShapeDtypeStruct((npad, d), jnp.float32),
    )(emap, amap, xs, W12, W12, W3)

    w_rep = jnp.broadcast_to(w2d.reshape(na)[:, None], (na, 16))
    y = _sc_combine(out_s, pos_flat, w_rep)
    return y.reshape(b, t_len, d)


def _sc_dispatch(xt, pos_flat, npad):
    tt, d = xt.shape
    na = pos_flat.shape[0]
    nw = 32
    npb = na // nw
    ch = min(64, npb)
    mesh = plsc.VectorSubcoreMesh(core_axis_name="c", subcore_axis_name="s")

    @functools.partial(
        pl.kernel, mesh=mesh,
        out_type=jax.ShapeDtypeStruct((npad, d), jnp.int32),
        scratch_types=[
            pltpu.VMEM((ch,), jnp.int32),
            pltpu.VMEM((ch, d), jnp.int32),
            pltpu.SemaphoreType.DMA,
        ],
    )
    def dispatch(x_hbm, pos_hbm, xs_hbm, idx_v, rows_v, sem):
        wid = lax.axis_index("s") * 2 + lax.axis_index("c")
        base = wid * npb
        tok0 = lax.rem(base, tt)
        for c in range(npb // ch):
            pltpu.sync_copy(pos_hbm.at[pl.ds(base + c * ch, ch)], idx_v)
            pltpu.sync_copy(x_hbm.at[pl.ds(tok0 + c * ch, ch)], rows_v)
            pltpu.async_copy(rows_v, xs_hbm.at[idx_v], sem).wait()

    return dispatch(xt, pos_flat)


def _sc_combine(out_s, pos_flat, w_rep):
    npad, d = out_s.shape
    na = pos_flat.shape[0]
    tt = na // TOP_K
    nw = 32
    tpw = tt // nw
    ch = min(32, tpw)
    nc16 = d // 16
    mesh = plsc.VectorSubcoreMesh(core_axis_name="c", subcore_axis_name="s")

    @functools.partial(
        pl.kernel, mesh=mesh,
        out_type=jax.ShapeDtypeStruct((tt, d), jnp.float32),
        scratch_types=[
            pltpu.VMEM((ch,), jnp.int32),
            pltpu.VMEM((ch,), jnp.int32),
            pltpu.VMEM((ch, 16), jnp.float32),
            pltpu.VMEM((ch, 16), jnp.float32),
            pltpu.VMEM((ch, d), jnp.float32),
            pltpu.VMEM((ch, d), jnp.float32),
            pltpu.SemaphoreType.DMA,
        ],
    )
    def combine(outs_hbm, pos_hbm, w_hbm, y_hbm,
                p0_v, p1_v, w0_v, w1_v, r0_v, r1_v, sem):
        wid = lax.axis_index("s") * 2 + lax.axis_index("c")
        tbase = wid * tpw
        for c in range(tpw // ch):
            t0 = tbase + c * ch
            pltpu.sync_copy(pos_hbm.at[pl.ds(t0, ch)], p0_v)
            pltpu.sync_copy(pos_hbm.at[pl.ds(tt + t0, ch)], p1_v)
            pltpu.sync_copy(w_hbm.at[pl.ds(t0, ch)], w0_v)
            pltpu.sync_copy(w_hbm.at[pl.ds(tt + t0, ch)], w1_v)
            pltpu.async_copy(outs_hbm.at[p0_v], r0_v, sem).wait()
            pltpu.async_copy(outs_hbm.at[p1_v], r1_v, sem).wait()

            def body(i, carry):
                w0b = w0_v[i, :]
                w1b = w1_v[i, :]
                for col in range(nc16):
                    sl = pl.ds(col * 16, 16)
                    r0_v[i, sl] = w0b * r0_v[i, sl] + w1b * r1_v[i, sl]
                return carry

            lax.fori_loop(0, ch, body, 0)
            pltpu.sync_copy(r0_v, y_hbm.at[pl.ds(t0, ch)])

    return combine(out_s, pos_flat, w_rep)


# speedup vs baseline: 1.8045x; 1.8045x over previous
import functools

import jax
import jax.numpy as jnp
from jax import lax
from jax.experimental import pallas as pl
from jax.experimental.pallas import tpu as pltpu
from jax.experimental.pallas import tpu_sc as plsc

TOP_K = 2
BLK = 256
HB = 1024


def _routing_body(x_ref, wg_ref, pos_ref, w_ref, emap_ref, amap_ref, *, n_e, blk):
    t_len = x_ref.shape[0]
    s = jax.lax.dot_general(wg_ref[...], x_ref[...], (((1,), (1,)), ((), ())),
                            preferred_element_type=jnp.float32)
    ei = jax.lax.broadcasted_iota(jnp.int32, s.shape, 0)
    v1 = jnp.max(s, axis=0, keepdims=True)
    i1 = jnp.min(jnp.where(s == v1, ei, n_e), axis=0, keepdims=True)
    s_m = jnp.where(ei == i1, -jnp.inf, s)
    v2 = jnp.max(s_m, axis=0, keepdims=True)
    i2 = jnp.min(jnp.where(s_m == v2, ei, n_e), axis=0, keepdims=True)
    p1 = 1.0 / (1.0 + jnp.exp(v2 - v1))
    w_ref[...] = jnp.concatenate([p1, 1.0 - p1], axis=0)

    e2d = jnp.concatenate([i1, i2], axis=0)
    tri = (jax.lax.broadcasted_iota(jnp.int32, (t_len, t_len), 0)
           <= jax.lax.broadcasted_iota(jnp.int32, (t_len, t_len), 1)
           ).astype(jnp.float32)
    ms = [(e2d == e).astype(jnp.float32) for e in range(n_e)]
    m_all = jnp.concatenate(ms, axis=0)
    c_all = jax.lax.dot_general(m_all, tri, (((1,), (0,)), ((), ())),
                                preferred_element_type=jnp.float32)

    rank = jnp.zeros_like(w_ref[...])
    counts = []
    for e in range(n_e):
        m = ms[e]
        c = c_all[2 * e:2 * e + 2]
        row0_tot = c[0:1, t_len - 1:t_len]
        carry = jnp.concatenate(
            [jnp.zeros((1, t_len), jnp.float32),
             jnp.broadcast_to(row0_tot, (1, t_len))], axis=0)
        rank = rank + m * (c - m + carry)
        counts.append(row0_tot + c[1:2, t_len - 1:t_len])

    blkstart = []
    run = jnp.zeros((1, 1), jnp.float32)
    for e in range(n_e):
        blkstart.append(run)
        run = run + jnp.floor((counts[e] + (blk - 1)) / blk)
    total_blocks = run

    astart = jnp.zeros_like(rank)
    for e in range(n_e):
        astart = astart + ms[e] * (blkstart[e] * blk)
    pos_ref[...] = (rank + astart).astype(jnp.int32)

    bi = jax.lax.broadcasted_iota(jnp.int32, (1, 128), 1)
    em = jnp.zeros((1, 128), jnp.int32)
    for e in range(n_e):
        bs = jnp.broadcast_to(blkstart[e].astype(jnp.int32), (1, 128))
        em = em + (bi >= bs).astype(jnp.int32)
    emap_ref[...] = em - 1
    amap_ref[...] = (
        bi < jnp.broadcast_to(total_blocks.astype(jnp.int32), (1, 128))
    ).astype(jnp.int32)


def _gemm_body(emap_ref, amap_ref, xs_ref, w1_ref, w2_ref, w3_ref, out_ref,
               acc_ref, *, blk):
    h = pl.program_id(0)
    b = pl.program_id(1)
    nh = pl.num_programs(0)

    @pl.when(amap_ref[0, b] > 0)
    def _compute():
        x = xs_ref[...]
        h1 = jax.lax.dot_general(x, w1_ref[0], (((1,), (1,)), ((), ())),
                                 preferred_element_type=jnp.float32)
        h2 = jax.lax.dot_general(x, w2_ref[0], (((1,), (1,)), ((), ())),
                                 preferred_element_type=jnp.float32)
        hidden = h1 * jax.lax.logistic(h1) * h2
        eo = jax.lax.dot_general(hidden, w3_ref[0], (((1,), (1,)), ((), ())),
                                 preferred_element_type=jnp.float32)

        sl = pl.ds(b * blk, blk)

        @pl.when(h == 0)
        def _init():
            acc_ref[sl, :] = eo

        @pl.when(h != 0)
        def _acc():
            acc_ref[sl, :] += eo

        @pl.when(h == nh - 1)
        def _fin():
            out_ref[...] = acc_ref[sl, :]


def kernel(x, Wg, W12, W3):
    b, t_len, d = x.shape
    n_e, h2, _ = W12.shape
    hdim = h2 // 2
    xt = x.reshape(b * t_len, d)
    tt = b * t_len
    na = TOP_K * tt
    blk = min(BLK, tt)
    nblk = na // blk + n_e
    npad = nblk * blk
    hb = min(HB, hdim)
    nh = hdim // hb

    pos2d, w2d, emap, amap = pl.pallas_call(
        functools.partial(_routing_body, n_e=n_e, blk=blk),
        out_shape=[
            jax.ShapeDtypeStruct((TOP_K, tt), jnp.int32),
            jax.ShapeDtypeStruct((TOP_K, tt), jnp.float32),
            jax.ShapeDtypeStruct((1, 128), jnp.int32),
            jax.ShapeDtypeStruct((1, 128), jnp.int32),
        ],
    )(xt, Wg)

    pos_flat = pos2d.reshape(na)
    xs = _sc_dispatch(xt, pos_flat, npad)
    out_s = pl.pallas_call(
        functools.partial(_gemm_body, blk=blk),
        grid_spec=pltpu.PrefetchScalarGridSpec(
            num_scalar_prefetch=2,
            grid=(nh, nblk),
            in_specs=[
                pl.BlockSpec((blk, d), lambda hh, bb, em, am: (bb, 0)),
                pl.BlockSpec((1, hb, d), lambda hh, bb, em, am: (em[0, bb], hh, 0)),
                pl.BlockSpec((1, hb, d),
                             lambda hh, bb, em, am, _nh=nh: (em[0, bb], _nh + hh, 0)),
                pl.BlockSpec((1, d, hb), lambda hh, bb, em, am: (em[0, bb], 0, hh)),
            ],
            out_specs=pl.BlockSpec((blk, d), lambda hh, bb, em, am: (bb, 0)),
            scratch_shapes=[pltpu.VMEM((npad, d), jnp.float32)],
        ),
        out_shape=jax.ShapeDtypeStruct((npad, d), jnp.float32),
    )(emap, amap, xs, W12, W12, W3)

    w_rep = jnp.broadcast_to(w2d.reshape(na)[:, None], (na, 16))
    y = _sc_combine(out_s, pos_flat, w_rep)
    return y.reshape(b, t_len, d)


def _sc_dispatch(xt, pos_flat, npad):
    tt, d = xt.shape
    na = pos_flat.shape[0]
    nw = 32
    npb = na // nw
    ch = min(64, npb)
    mesh = plsc.VectorSubcoreMesh(core_axis_name="c", subcore_axis_name="s")

    @functools.partial(
        pl.kernel, mesh=mesh,
        out_type=jax.ShapeDtypeStruct((npad, d), jnp.float32),
        scratch_types=[
            pltpu.VMEM((ch,), jnp.int32),
            pltpu.VMEM((ch, d), jnp.float32),
            pltpu.SemaphoreType.DMA,
        ],
    )
    def dispatch(x_hbm, pos_hbm, xs_hbm, idx_v, rows_v, sem):
        wid = lax.axis_index("s") * 2 + lax.axis_index("c")
        base = wid * npb
        tok0 = lax.rem(base, tt)
        for c in range(npb // ch):
            pltpu.sync_copy(pos_hbm.at[pl.ds(base + c * ch, ch)], idx_v)
            pltpu.sync_copy(x_hbm.at[pl.ds(tok0 + c * ch, ch)], rows_v)
            pltpu.async_copy(rows_v, xs_hbm.at[idx_v], sem).wait()

    return dispatch(xt, pos_flat)


def _sc_combine(out_s, pos_flat, w_rep):
    npad, d = out_s.shape
    na = pos_flat.shape[0]
    tt = na // TOP_K
    nw = 32
    tpw = tt // nw
    ch = min(32, tpw)
    nc16 = d // 16
    mesh = plsc.VectorSubcoreMesh(core_axis_name="c", subcore_axis_name="s")

    @functools.partial(
        pl.kernel, mesh=mesh,
        out_type=jax.ShapeDtypeStruct((tt, d), jnp.float32),
        scratch_types=[
            pltpu.VMEM((ch,), jnp.int32),
            pltpu.VMEM((ch,), jnp.int32),
            pltpu.VMEM((ch, 16), jnp.float32),
            pltpu.VMEM((ch, 16), jnp.float32),
            pltpu.VMEM((ch, d), jnp.float32),
            pltpu.VMEM((ch, d), jnp.float32),
            pltpu.SemaphoreType.DMA,
        ],
    )
    def combine(outs_hbm, pos_hbm, w_hbm, y_hbm,
                p0_v, p1_v, w0_v, w1_v, r0_v, r1_v, sem):
        wid = lax.axis_index("s") * 2 + lax.axis_index("c")
        tbase = wid * tpw
        for c in range(tpw // ch):
            t0 = tbase + c * ch
            pltpu.sync_copy(pos_hbm.at[pl.ds(t0, ch)], p0_v)
            pltpu.sync_copy(pos_hbm.at[pl.ds(tt + t0, ch)], p1_v)
            pltpu.sync_copy(w_hbm.at[pl.ds(t0, ch)], w0_v)
            pltpu.sync_copy(w_hbm.at[pl.ds(tt + t0, ch)], w1_v)
            pltpu.async_copy(outs_hbm.at[p0_v], r0_v, sem).wait()
            pltpu.async_copy(outs_hbm.at[p1_v], r1_v, sem).wait()

            def body(i, carry):
                w0b = w0_v[i, :]
                w1b = w1_v[i, :]
                for col in range(nc16):
                    sl = pl.ds(col * 16, 16)
                    r0_v[i, sl] = w0b * r0_v[i, sl] + w1b * r1_v[i, sl]
                return carry

            lax.fori_loop(0, ch, body, 0)
            pltpu.sync_copy(r0_v, y_hbm.at[pl.ds(t0, ch)])

    return combine(out_s, pos_flat, w_rep)


# speedup vs baseline: 2.0509x; 1.1365x over previous
import functools

import jax
import jax.numpy as jnp
from jax import lax
from jax.experimental import pallas as pl
from jax.experimental.pallas import tpu as pltpu
from jax.experimental.pallas import tpu_sc as plsc

TOP_K = 2
BLK = 256
HB = 2048


def _routing_body(x_ref, wg_ref, pos_ref, w_ref, emap_ref, amap_ref, *, n_e, blk):
    t_len = x_ref.shape[0]
    s = jax.lax.dot_general(wg_ref[...], x_ref[...], (((1,), (1,)), ((), ())),
                            preferred_element_type=jnp.float32)
    ei = jax.lax.broadcasted_iota(jnp.int32, s.shape, 0)
    v1 = jnp.max(s, axis=0, keepdims=True)
    i1 = jnp.min(jnp.where(s == v1, ei, n_e), axis=0, keepdims=True)
    s_m = jnp.where(ei == i1, -jnp.inf, s)
    v2 = jnp.max(s_m, axis=0, keepdims=True)
    i2 = jnp.min(jnp.where(s_m == v2, ei, n_e), axis=0, keepdims=True)
    p1 = 1.0 / (1.0 + jnp.exp(v2 - v1))
    w_ref[...] = jnp.concatenate([p1, 1.0 - p1], axis=0)

    e2d = jnp.concatenate([i1, i2], axis=0)
    tri = (jax.lax.broadcasted_iota(jnp.int32, (t_len, t_len), 0)
           <= jax.lax.broadcasted_iota(jnp.int32, (t_len, t_len), 1)
           ).astype(jnp.float32)
    ms = [(e2d == e).astype(jnp.float32) for e in range(n_e)]
    m_all = jnp.concatenate(ms, axis=0)
    c_all = jax.lax.dot_general(m_all, tri, (((1,), (0,)), ((), ())),
                                preferred_element_type=jnp.float32)

    rank = jnp.zeros_like(w_ref[...])
    counts = []
    for e in range(n_e):
        m = ms[e]
        c = c_all[2 * e:2 * e + 2]
        row0_tot = c[0:1, t_len - 1:t_len]
        carry = jnp.concatenate(
            [jnp.zeros((1, t_len), jnp.float32),
             jnp.broadcast_to(row0_tot, (1, t_len))], axis=0)
        rank = rank + m * (c - m + carry)
        counts.append(row0_tot + c[1:2, t_len - 1:t_len])

    blkstart = []
    run = jnp.zeros((1, 1), jnp.float32)
    for e in range(n_e):
        blkstart.append(run)
        run = run + jnp.floor((counts[e] + (blk - 1)) / blk)
    total_blocks = run

    astart = jnp.zeros_like(rank)
    for e in range(n_e):
        astart = astart + ms[e] * (blkstart[e] * blk)
    pos_ref[...] = (rank + astart).astype(jnp.int32)

    bi = jax.lax.broadcasted_iota(jnp.int32, (1, 128), 1)
    em = jnp.zeros((1, 128), jnp.int32)
    for e in range(n_e):
        bs = jnp.broadcast_to(blkstart[e].astype(jnp.int32), (1, 128))
        em = em + (bi >= bs).astype(jnp.int32)
    emap_ref[...] = em - 1
    amap_ref[...] = (
        bi < jnp.broadcast_to(total_blocks.astype(jnp.int32), (1, 128))
    ).astype(jnp.int32)


def _gemm_body(emap_ref, amap_ref, xs_ref, w1_ref, w2_ref, w3_ref, out_ref,
               acc_ref, *, blk):
    h = pl.program_id(0)
    b = pl.program_id(1)
    nh = pl.num_programs(0)

    @pl.when(amap_ref[0, b] > 0)
    def _compute():
        x = xs_ref[...]
        h1 = jax.lax.dot_general(x, w1_ref[0], (((1,), (1,)), ((), ())),
                                 preferred_element_type=jnp.float32)
        h2 = jax.lax.dot_general(x, w2_ref[0], (((1,), (1,)), ((), ())),
                                 preferred_element_type=jnp.float32)
        hidden = h1 * jax.lax.logistic(h1) * h2
        eo = jax.lax.dot_general(hidden, w3_ref[0], (((1,), (1,)), ((), ())),
                                 preferred_element_type=jnp.float32)

        sl = pl.ds(b * blk, blk)

        @pl.when(h == 0)
        def _init():
            acc_ref[sl, :] = eo

        @pl.when(h != 0)
        def _acc():
            acc_ref[sl, :] += eo

        @pl.when(h == nh - 1)
        def _fin():
            out_ref[...] = acc_ref[sl, :]


def kernel(x, Wg, W12, W3):
    b, t_len, d = x.shape
    n_e, h2, _ = W12.shape
    hdim = h2 // 2
    xt = x.reshape(b * t_len, d)
    tt = b * t_len
    na = TOP_K * tt
    blk = min(BLK, tt)
    nblk = na // blk + n_e
    npad = nblk * blk
    hb = min(HB, hdim)
    nh = hdim // hb

    pos2d, w2d, emap, amap = pl.pallas_call(
        functools.partial(_routing_body, n_e=n_e, blk=blk),
        out_shape=[
            jax.ShapeDtypeStruct((TOP_K, tt), jnp.int32),
            jax.ShapeDtypeStruct((TOP_K, tt), jnp.float32),
            jax.ShapeDtypeStruct((1, 128), jnp.int32),
            jax.ShapeDtypeStruct((1, 128), jnp.int32),
        ],
    )(xt, Wg)

    pos_flat = pos2d.reshape(na)
    xs = _sc_dispatch(xt, pos_flat, npad)
    out_s = pl.pallas_call(
        functools.partial(_gemm_body, blk=blk),
        grid_spec=pltpu.PrefetchScalarGridSpec(
            num_scalar_prefetch=2,
            grid=(nh, nblk),
            in_specs=[
                pl.BlockSpec((blk, d), lambda hh, bb, em, am: (bb, 0)),
                pl.BlockSpec((1, hb, d), lambda hh, bb, em, am: (em[0, bb], hh, 0)),
                pl.BlockSpec((1, hb, d),
                             lambda hh, bb, em, am, _nh=nh: (em[0, bb], _nh + hh, 0)),
                pl.BlockSpec((1, d, hb), lambda hh, bb, em, am: (em[0, bb], 0, hh)),
            ],
            out_specs=pl.BlockSpec((blk, d), lambda hh, bb, em, am: (bb, 0)),
            scratch_shapes=[pltpu.VMEM((npad, d), jnp.float32)],
        ),
        out_shape=jax.ShapeDtypeStruct((npad, d), jnp.float32),
        compiler_params=pltpu.CompilerParams(
            vmem_limit_bytes=100 * 1024 * 1024),
    )(emap, amap, xs, W12, W12, W3)

    w_rep = jnp.broadcast_to(w2d.reshape(na)[:, None], (na, 16))
    y = _sc_combine(out_s, pos_flat, w_rep)
    return y.reshape(b, t_len, d)


def _sc_dispatch(xt, pos_flat, npad):
    tt, d = xt.shape
    na = pos_flat.shape[0]
    nw = 32
    npb = na // nw
    ch = min(64, npb)
    mesh = plsc.VectorSubcoreMesh(core_axis_name="c", subcore_axis_name="s")

    @functools.partial(
        pl.kernel, mesh=mesh,
        out_type=jax.ShapeDtypeStruct((npad, d), jnp.float32),
        scratch_types=[
            pltpu.VMEM((ch,), jnp.int32),
            pltpu.VMEM((ch, d), jnp.float32),
            pltpu.SemaphoreType.DMA,
        ],
    )
    def dispatch(x_hbm, pos_hbm, xs_hbm, idx_v, rows_v, sem):
        wid = lax.axis_index("s") * 2 + lax.axis_index("c")
        base = wid * npb
        tok0 = lax.rem(base, tt)
        for c in range(npb // ch):
            pltpu.sync_copy(pos_hbm.at[pl.ds(base + c * ch, ch)], idx_v)
            pltpu.sync_copy(x_hbm.at[pl.ds(tok0 + c * ch, ch)], rows_v)
            pltpu.async_copy(rows_v, xs_hbm.at[idx_v], sem).wait()

    return dispatch(xt, pos_flat)


def _sc_combine(out_s, pos_flat, w_rep):
    npad, d = out_s.shape
    na = pos_flat.shape[0]
    tt = na // TOP_K
    nw = 32
    tpw = tt // nw
    ch = min(32, tpw)
    nc16 = d // 16
    mesh = plsc.VectorSubcoreMesh(core_axis_name="c", subcore_axis_name="s")

    @functools.partial(
        pl.kernel, mesh=mesh,
        out_type=jax.ShapeDtypeStruct((tt, d), jnp.float32),
        scratch_types=[
            pltpu.VMEM((ch,), jnp.int32),
            pltpu.VMEM((ch,), jnp.int32),
            pltpu.VMEM((ch, 16), jnp.float32),
            pltpu.VMEM((ch, 16), jnp.float32),
            pltpu.VMEM((ch, d), jnp.float32),
            pltpu.VMEM((ch, d), jnp.float32),
            pltpu.SemaphoreType.DMA,
        ],
    )
    def combine(outs_hbm, pos_hbm, w_hbm, y_hbm,
                p0_v, p1_v, w0_v, w1_v, r0_v, r1_v, sem):
        wid = lax.axis_index("s") * 2 + lax.axis_index("c")
        tbase = wid * tpw
        for c in range(tpw // ch):
            t0 = tbase + c * ch
            pltpu.sync_copy(pos_hbm.at[pl.ds(t0, ch)], p0_v)
            pltpu.sync_copy(pos_hbm.at[pl.ds(tt + t0, ch)], p1_v)
            pltpu.sync_copy(w_hbm.at[pl.ds(t0, ch)], w0_v)
            pltpu.sync_copy(w_hbm.at[pl.ds(tt + t0, ch)], w1_v)
            pltpu.async_copy(outs_hbm.at[p0_v], r0_v, sem).wait()
            pltpu.async_copy(outs_hbm.at[p1_v], r1_v, sem).wait()

            def body(i, carry):
                w0b = w0_v[i, :]
                w1b = w1_v[i, :]
                for col in range(nc16):
                    sl = pl.ds(col * 16, 16)
                    r0_v[i, sl] = w0b * r0_v[i, sl] + w1b * r1_v[i, sl]
                return carry

            lax.fori_loop(0, ch, body, 0)
            pltpu.sync_copy(r0_v, y_hbm.at[pl.ds(t0, ch)])

    return combine(out_s, pos_flat, w_rep)


# speedup vs baseline: 2.2427x; 1.0935x over previous
import functools

import jax
import jax.numpy as jnp
from jax import lax
from jax.experimental import pallas as pl
from jax.experimental.pallas import tpu as pltpu
from jax.experimental.pallas import tpu_sc as plsc

TOP_K = 2
BLK = 512
HB = 2048


def _routing_body(x_ref, wg_ref, pos_ref, w_ref, emap_ref, amap_ref, *, n_e, blk):
    t_len = x_ref.shape[0]
    s = jax.lax.dot_general(wg_ref[...], x_ref[...], (((1,), (1,)), ((), ())),
                            preferred_element_type=jnp.float32)
    ei = jax.lax.broadcasted_iota(jnp.int32, s.shape, 0)
    v1 = jnp.max(s, axis=0, keepdims=True)
    i1 = jnp.min(jnp.where(s == v1, ei, n_e), axis=0, keepdims=True)
    s_m = jnp.where(ei == i1, -jnp.inf, s)
    v2 = jnp.max(s_m, axis=0, keepdims=True)
    i2 = jnp.min(jnp.where(s_m == v2, ei, n_e), axis=0, keepdims=True)
    p1 = 1.0 / (1.0 + jnp.exp(v2 - v1))
    w_ref[...] = jnp.concatenate([p1, 1.0 - p1], axis=0)

    e2d = jnp.concatenate([i1, i2], axis=0)
    tri = (jax.lax.broadcasted_iota(jnp.int32, (t_len, t_len), 0)
           <= jax.lax.broadcasted_iota(jnp.int32, (t_len, t_len), 1)
           ).astype(jnp.float32)
    ms = [(e2d == e).astype(jnp.float32) for e in range(n_e)]
    m_all = jnp.concatenate(ms, axis=0)
    c_all = jax.lax.dot_general(m_all, tri, (((1,), (0,)), ((), ())),
                                preferred_element_type=jnp.float32)

    rank = jnp.zeros_like(w_ref[...])
    counts = []
    for e in range(n_e):
        m = ms[e]
        c = c_all[2 * e:2 * e + 2]
        row0_tot = c[0:1, t_len - 1:t_len]
        carry = jnp.concatenate(
            [jnp.zeros((1, t_len), jnp.float32),
             jnp.broadcast_to(row0_tot, (1, t_len))], axis=0)
        rank = rank + m * (c - m + carry)
        counts.append(row0_tot + c[1:2, t_len - 1:t_len])

    blkstart = []
    run = jnp.zeros((1, 1), jnp.float32)
    for e in range(n_e):
        blkstart.append(run)
        run = run + jnp.floor((counts[e] + (blk - 1)) / blk)
    total_blocks = run

    astart = jnp.zeros_like(rank)
    for e in range(n_e):
        astart = astart + ms[e] * (blkstart[e] * blk)
    pos_ref[...] = (rank + astart).astype(jnp.int32)

    bi = jax.lax.broadcasted_iota(jnp.int32, (1, 128), 1)
    em = jnp.zeros((1, 128), jnp.int32)
    for e in range(n_e):
        bs = jnp.broadcast_to(blkstart[e].astype(jnp.int32), (1, 128))
        em = em + (bi >= bs).astype(jnp.int32)
    emap_ref[...] = em - 1
    amap_ref[...] = (
        bi < jnp.broadcast_to(total_blocks.astype(jnp.int32), (1, 128))
    ).astype(jnp.int32)


def _gemm_body(emap_ref, amap_ref, xs_ref, w1_ref, w2_ref, w3_ref, out_ref,
               acc_ref, *, blk):
    h = pl.program_id(0)
    b = pl.program_id(1)
    nh = pl.num_programs(0)

    @pl.when(amap_ref[0, b] > 0)
    def _compute():
        x = xs_ref[...]
        h1 = jax.lax.dot_general(x, w1_ref[0], (((1,), (1,)), ((), ())),
                                 preferred_element_type=jnp.float32)
        h2 = jax.lax.dot_general(x, w2_ref[0], (((1,), (1,)), ((), ())),
                                 preferred_element_type=jnp.float32)
        hidden = h1 * jax.lax.logistic(h1) * h2
        eo = jax.lax.dot_general(hidden, w3_ref[0], (((1,), (1,)), ((), ())),
                                 preferred_element_type=jnp.float32)

        sl = pl.ds(b * blk, blk)

        @pl.when(h == 0)
        def _init():
            acc_ref[sl, :] = eo

        @pl.when(h != 0)
        def _acc():
            acc_ref[sl, :] += eo

        @pl.when(h == nh - 1)
        def _fin():
            out_ref[...] = acc_ref[sl, :]


def kernel(x, Wg, W12, W3):
    b, t_len, d = x.shape
    n_e, h2, _ = W12.shape
    hdim = h2 // 2
    xt = x.reshape(b * t_len, d)
    tt = b * t_len
    na = TOP_K * tt
    blk = min(BLK, tt)
    nblk = na // blk + n_e
    npad = nblk * blk
    hb = min(HB, hdim)
    nh = hdim // hb

    pos2d, w2d, emap, amap = pl.pallas_call(
        functools.partial(_routing_body, n_e=n_e, blk=blk),
        out_shape=[
            jax.ShapeDtypeStruct((TOP_K, tt), jnp.int32),
            jax.ShapeDtypeStruct((TOP_K, tt), jnp.float32),
            jax.ShapeDtypeStruct((1, 128), jnp.int32),
            jax.ShapeDtypeStruct((1, 128), jnp.int32),
        ],
    )(xt, Wg)

    pos_flat = pos2d.reshape(na)
    xs = _sc_dispatch(xt, pos_flat, npad)
    out_s = pl.pallas_call(
        functools.partial(_gemm_body, blk=blk),
        grid_spec=pltpu.PrefetchScalarGridSpec(
            num_scalar_prefetch=2,
            grid=(nh, nblk),
            in_specs=[
                pl.BlockSpec((blk, d), lambda hh, bb, em, am: (bb, 0)),
                pl.BlockSpec((1, hb, d), lambda hh, bb, em, am: (em[0, bb], hh, 0)),
                pl.BlockSpec((1, hb, d),
                             lambda hh, bb, em, am, _nh=nh: (em[0, bb], _nh + hh, 0)),
                pl.BlockSpec((1, d, hb), lambda hh, bb, em, am: (em[0, bb], 0, hh)),
            ],
            out_specs=pl.BlockSpec((blk, d), lambda hh, bb, em, am: (bb, 0)),
            scratch_shapes=[pltpu.VMEM((npad, d), jnp.float32)],
        ),
        out_shape=jax.ShapeDtypeStruct((npad, d), jnp.float32),
        compiler_params=pltpu.CompilerParams(
            vmem_limit_bytes=100 * 1024 * 1024),
    )(emap, amap, xs, W12, W12, W3)

    w_rep = jnp.broadcast_to(w2d.reshape(na)[:, None], (na, 16))
    y = _sc_combine(out_s, pos_flat, w_rep)
    return y.reshape(b, t_len, d)


def _sc_dispatch(xt, pos_flat, npad):
    tt, d = xt.shape
    na = pos_flat.shape[0]
    nw = 32
    npb = na // nw
    ch = min(64, npb)
    mesh = plsc.VectorSubcoreMesh(core_axis_name="c", subcore_axis_name="s")

    @functools.partial(
        pl.kernel, mesh=mesh,
        out_type=jax.ShapeDtypeStruct((npad, d), jnp.float32),
        scratch_types=[
            pltpu.VMEM((ch,), jnp.int32),
            pltpu.VMEM((ch, d), jnp.float32),
            pltpu.SemaphoreType.DMA,
        ],
    )
    def dispatch(x_hbm, pos_hbm, xs_hbm, idx_v, rows_v, sem):
        wid = lax.axis_index("s") * 2 + lax.axis_index("c")
        base = wid * npb
        tok0 = lax.rem(base, tt)
        for c in range(npb // ch):
            pltpu.sync_copy(pos_hbm.at[pl.ds(base + c * ch, ch)], idx_v)
            pltpu.sync_copy(x_hbm.at[pl.ds(tok0 + c * ch, ch)], rows_v)
            pltpu.async_copy(rows_v, xs_hbm.at[idx_v], sem).wait()

    return dispatch(xt, pos_flat)


def _sc_combine(out_s, pos_flat, w_rep):
    npad, d = out_s.shape
    na = pos_flat.shape[0]
    tt = na // TOP_K
    nw = 32
    tpw = tt // nw
    ch = min(32, tpw)
    nc16 = d // 16
    mesh = plsc.VectorSubcoreMesh(core_axis_name="c", subcore_axis_name="s")

    @functools.partial(
        pl.kernel, mesh=mesh,
        out_type=jax.ShapeDtypeStruct((tt, d), jnp.float32),
        scratch_types=[
            pltpu.VMEM((ch,), jnp.int32),
            pltpu.VMEM((ch,), jnp.int32),
            pltpu.VMEM((ch, 16), jnp.float32),
            pltpu.VMEM((ch, 16), jnp.float32),
            pltpu.VMEM((ch, d), jnp.float32),
            pltpu.VMEM((ch, d), jnp.float32),
            pltpu.SemaphoreType.DMA,
        ],
    )
    def combine(outs_hbm, pos_hbm, w_hbm, y_hbm,
                p0_v, p1_v, w0_v, w1_v, r0_v, r1_v, sem):
        wid = lax.axis_index("s") * 2 + lax.axis_index("c")
        tbase = wid * tpw
        for c in range(tpw // ch):
            t0 = tbase + c * ch
            pltpu.sync_copy(pos_hbm.at[pl.ds(t0, ch)], p0_v)
            pltpu.sync_copy(pos_hbm.at[pl.ds(tt + t0, ch)], p1_v)
            pltpu.sync_copy(w_hbm.at[pl.ds(t0, ch)], w0_v)
            pltpu.sync_copy(w_hbm.at[pl.ds(tt + t0, ch)], w1_v)
            pltpu.async_copy(outs_hbm.at[p0_v], r0_v, sem).wait()
            pltpu.async_copy(outs_hbm.at[p1_v], r1_v, sem).wait()

            def body(i, carry):
                w0b = w0_v[i, :]
                w1b = w1_v[i, :]
                for col in range(nc16):
                    sl = pl.ds(col * 16, 16)
                    r0_v[i, sl] = w0b * r0_v[i, sl] + w1b * r1_v[i, sl]
                return carry

            lax.fori_loop(0, ch, body, 0)
            pltpu.sync_copy(r0_v, y_hbm.at[pl.ds(t0, ch)])

    return combine(out_s, pos_flat, w_rep)


# speedup vs baseline: 2.2863x; 1.0194x over previous
import functools

import jax
import jax.numpy as jnp
from jax import lax
from jax.experimental import pallas as pl
from jax.experimental.pallas import tpu as pltpu
from jax.experimental.pallas import tpu_sc as plsc

TOP_K = 2
BLK = 512
HB = 2048


def _routing_body(x_ref, wg_ref, pos_ref, w_ref, emap_ref, amap_ref, *, n_e, blk):
    t_len = x_ref.shape[0]
    s = jax.lax.dot_general(wg_ref[...], x_ref[...], (((1,), (1,)), ((), ())),
                            preferred_element_type=jnp.float32)
    ei = jax.lax.broadcasted_iota(jnp.int32, s.shape, 0)
    v1 = jnp.max(s, axis=0, keepdims=True)
    i1 = jnp.min(jnp.where(s == v1, ei, n_e), axis=0, keepdims=True)
    s_m = jnp.where(ei == i1, -jnp.inf, s)
    v2 = jnp.max(s_m, axis=0, keepdims=True)
    i2 = jnp.min(jnp.where(s_m == v2, ei, n_e), axis=0, keepdims=True)
    p1 = 1.0 / (1.0 + jnp.exp(v2 - v1))
    w_ref[...] = jnp.concatenate([p1, 1.0 - p1], axis=0)

    e2d = jnp.concatenate([i1, i2], axis=0)
    tri = (jax.lax.broadcasted_iota(jnp.int32, (t_len, t_len), 0)
           <= jax.lax.broadcasted_iota(jnp.int32, (t_len, t_len), 1)
           ).astype(jnp.float32)
    ms = [(e2d == e).astype(jnp.float32) for e in range(n_e)]
    m_all = jnp.concatenate(ms, axis=0)
    c_all = jax.lax.dot_general(m_all, tri, (((1,), (0,)), ((), ())),
                                preferred_element_type=jnp.float32)

    rank = jnp.zeros_like(w_ref[...])
    counts = []
    for e in range(n_e):
        m = ms[e]
        c = c_all[2 * e:2 * e + 2]
        row0_tot = c[0:1, t_len - 1:t_len]
        carry = jnp.concatenate(
            [jnp.zeros((1, t_len), jnp.float32),
             jnp.broadcast_to(row0_tot, (1, t_len))], axis=0)
        rank = rank + m * (c - m + carry)
        counts.append(row0_tot + c[1:2, t_len - 1:t_len])

    blkstart = []
    run = jnp.zeros((1, 1), jnp.float32)
    for e in range(n_e):
        blkstart.append(run)
        run = run + jnp.floor((counts[e] + (blk - 1)) / blk)
    total_blocks = run

    astart = jnp.zeros_like(rank)
    for e in range(n_e):
        astart = astart + ms[e] * (blkstart[e] * blk)
    pos_ref[...] = (rank + astart).astype(jnp.int32)

    bi = jax.lax.broadcasted_iota(jnp.int32, (1, 128), 1)
    em = jnp.zeros((1, 128), jnp.int32)
    for e in range(n_e):
        bs = jnp.broadcast_to(blkstart[e].astype(jnp.int32), (1, 128))
        em = em + (bi >= bs).astype(jnp.int32)
    emap_ref[...] = em - 1
    amap_ref[...] = (
        bi < jnp.broadcast_to(total_blocks.astype(jnp.int32), (1, 128))
    ).astype(jnp.int32)


def _gemm_body(emap_ref, amap_ref, xs_ref, w1_ref, w2_ref, w3_ref, out_ref,
               acc_ref, *, blk):
    h = pl.program_id(0)
    b = pl.program_id(1)
    nh = pl.num_programs(0)

    @pl.when(amap_ref[0, b] > 0)
    def _compute():
        x = xs_ref[...]
        h1 = jax.lax.dot_general(x, w1_ref[0], (((1,), (1,)), ((), ())),
                                 preferred_element_type=jnp.float32)
        h2 = jax.lax.dot_general(x, w2_ref[0], (((1,), (1,)), ((), ())),
                                 preferred_element_type=jnp.float32)
        hidden = h1 * jax.lax.logistic(h1) * h2
        eo = jax.lax.dot_general(hidden, w3_ref[0], (((1,), (1,)), ((), ())),
                                 preferred_element_type=jnp.float32)

        sl = pl.ds(b * blk, blk)

        @pl.when(h == 0)
        def _init():
            acc_ref[sl, :] = eo

        @pl.when(h != 0)
        def _acc():
            acc_ref[sl, :] += eo

        @pl.when(h == nh - 1)
        def _fin():
            out_ref[...] = acc_ref[sl, :]


def kernel(x, Wg, W12, W3):
    b, t_len, d = x.shape
    n_e, h2, _ = W12.shape
    hdim = h2 // 2
    xt = x.reshape(b * t_len, d)
    tt = b * t_len
    na = TOP_K * tt
    blk = min(BLK, tt)
    nblk = na // blk + n_e
    npad = nblk * blk
    hb = min(HB, hdim)
    nh = hdim // hb

    pos2d, w2d, emap, amap = pl.pallas_call(
        functools.partial(_routing_body, n_e=n_e, blk=blk),
        out_shape=[
            jax.ShapeDtypeStruct((TOP_K, tt), jnp.int32),
            jax.ShapeDtypeStruct((TOP_K, tt), jnp.float32),
            jax.ShapeDtypeStruct((1, 128), jnp.int32),
            jax.ShapeDtypeStruct((1, 128), jnp.int32),
        ],
    )(xt, Wg)

    pos_flat = pos2d.reshape(na)
    xs = _sc_dispatch(xt, pos_flat, npad)
    out_s = pl.pallas_call(
        functools.partial(_gemm_body, blk=blk),
        grid_spec=pltpu.PrefetchScalarGridSpec(
            num_scalar_prefetch=2,
            grid=(nh, nblk),
            in_specs=[
                pl.BlockSpec((blk, d), lambda hh, bb, em, am: (bb, 0)),
                pl.BlockSpec((1, hb, d), lambda hh, bb, em, am: (em[0, bb], hh, 0)),
                pl.BlockSpec((1, hb, d),
                             lambda hh, bb, em, am, _nh=nh: (em[0, bb], _nh + hh, 0)),
                pl.BlockSpec((1, d, hb), lambda hh, bb, em, am: (em[0, bb], 0, hh)),
            ],
            out_specs=pl.BlockSpec((blk, d), lambda hh, bb, em, am: (bb, 0)),
            scratch_shapes=[pltpu.VMEM((npad, d), jnp.float32)],
        ),
        out_shape=jax.ShapeDtypeStruct((npad, d), jnp.float32),
        compiler_params=pltpu.CompilerParams(
            vmem_limit_bytes=100 * 1024 * 1024),
    )(emap, amap, xs, W12, W12, W3)

    w_rep = jnp.broadcast_to(w2d.reshape(na)[:, None], (na, 16))
    y = _sc_combine(out_s, pos_flat, w_rep)
    return y.reshape(b, t_len, d)


def _sc_dispatch(xt, pos_flat, npad):
    tt, d = xt.shape
    na = pos_flat.shape[0]
    nw = 32
    npb = na // nw
    ch = min(64, npb)
    mesh = plsc.VectorSubcoreMesh(core_axis_name="c", subcore_axis_name="s")

    @functools.partial(
        pl.kernel, mesh=mesh,
        out_type=jax.ShapeDtypeStruct((npad, d), jnp.float32),
        scratch_types=[
            pltpu.VMEM((ch,), jnp.int32),
            pltpu.VMEM((ch, d), jnp.float32),
            pltpu.SemaphoreType.DMA,
        ],
    )
    def dispatch(x_hbm, pos_hbm, xs_hbm, idx_v, rows_v, sem):
        wid = lax.axis_index("s") * 2 + lax.axis_index("c")
        base = wid * npb
        tok0 = lax.rem(base, tt)
        for c in range(npb // ch):
            pltpu.sync_copy(pos_hbm.at[pl.ds(base + c * ch, ch)], idx_v)
            pltpu.sync_copy(x_hbm.at[pl.ds(tok0 + c * ch, ch)], rows_v)
            pltpu.async_copy(rows_v, xs_hbm.at[idx_v], sem).wait()

    return dispatch(xt, pos_flat)


def _sc_combine(out_s, pos_flat, w_rep):
    npad, d = out_s.shape
    na = pos_flat.shape[0]
    tt = na // TOP_K
    nw = 32
    tpw = tt // nw
    ch = min(16, tpw)
    nch = tpw // ch
    nc16 = d // 16
    mesh = plsc.VectorSubcoreMesh(core_axis_name="c", subcore_axis_name="s")

    @functools.partial(
        pl.kernel, mesh=mesh,
        out_type=jax.ShapeDtypeStruct((tt, d), jnp.float32),
        scratch_types=[
            pltpu.VMEM((tpw,), jnp.int32),
            pltpu.VMEM((tpw,), jnp.int32),
            pltpu.VMEM((tpw, 16), jnp.float32),
            pltpu.VMEM((tpw, 16), jnp.float32),
            pltpu.VMEM((2, ch, d), jnp.float32),
            pltpu.VMEM((2, ch, d), jnp.float32),
            pltpu.SemaphoreType.DMA,
            pltpu.SemaphoreType.DMA,
        ],
    )
    def combine(outs_hbm, pos_hbm, w_hbm, y_hbm,
                p0_v, p1_v, w0_v, w1_v, r0_v, r1_v, sem0, sem1):
        wid = lax.axis_index("s") * 2 + lax.axis_index("c")
        tbase = wid * tpw
        pltpu.sync_copy(pos_hbm.at[pl.ds(tbase, tpw)], p0_v)
        pltpu.sync_copy(pos_hbm.at[pl.ds(tt + tbase, tpw)], p1_v)
        pltpu.sync_copy(w_hbm.at[pl.ds(tbase, tpw)], w0_v)
        pltpu.sync_copy(w_hbm.at[pl.ds(tt + tbase, tpw)], w1_v)

        sems = [sem0, sem1]

        def start(c):
            buf = c % 2
            i0 = p0_v[pl.ds(c * ch, ch)]
            i1 = p1_v[pl.ds(c * ch, ch)]
            c0 = pltpu.async_copy(outs_hbm.at[i0], r0_v.at[buf], sems[buf])
            c1 = pltpu.async_copy(outs_hbm.at[i1], r1_v.at[buf], sems[buf])
            return c0, c1

        pend = start(0)
        for c in range(nch):
            buf = c % 2
            pend[0].wait()
            pend[1].wait()
            if c + 1 < nch:
                pend = start(c + 1)

            def body(i, carry):
                w0b = w0_v[c * ch + i, :]
                w1b = w1_v[c * ch + i, :]
                for col in range(nc16):
                    sl = pl.ds(col * 16, 16)
                    r0_v[buf, i, sl] = (w0b * r0_v[buf, i, sl]
                                        + w1b * r1_v[buf, i, sl])
                return carry

            lax.fori_loop(0, ch, body, 0)
            pltpu.sync_copy(r0_v.at[buf], y_hbm.at[pl.ds(tbase + c * ch, ch)])

    return combine(out_s, pos_flat, w_rep)
